# Initial kernel scaffold; baseline (speedup 1.0000x reference)
import jax
import jax.numpy as jnp
from jax.experimental import pallas as pl


def kernel(urls, emb_table, W_ih_f, W_hh_f, b_f, W_ih_b, W_hh_b, b_b, fc_w, fc_b, fc1_w, fc1_b, fc2_w, fc2_b, fc3_w, fc3_b):
    raise NotImplementedError("write your pallas kernel here")


# speedup vs baseline: 3.2370x; 3.2370x over previous
import functools

import jax
import jax.numpy as jnp
from jax.experimental import pallas as pl
from jax.experimental.pallas import tpu as pltpu

INPUT_DIM = 256
EMB_DIM = 128
HIDDEN_DIM = 512
BATCH = 128
SEQ = 200
H4 = 4 * HIDDEN_DIM


def _lstm_kernel(urls_ref, emb_ref, wihf_ref, whhf_ref, bf_ref,
                 wihb_ref, whhb_ref, bb_ref,
                 fcw_ref, fcb_ref, fc1w_ref, fc1b_ref,
                 fc2w_ref, fc2b_ref, fc3w_ref, fc3b_ref,
                 out_ref, aux_ref,
                 mf_scr, mb_scr, hf_scr, cf_scr, hb_scr, cb_scr):
    f32 = jnp.float32

    emb = emb_ref[...]
    mf_scr[...] = jnp.dot(emb, wihf_ref[...], preferred_element_type=f32) + bf_ref[...]
    mb_scr[...] = jnp.dot(emb, wihb_ref[...], preferred_element_type=f32) + bb_ref[...]

    zeros = jnp.zeros((BATCH, HIDDEN_DIM), f32)
    hf_scr[...] = zeros
    cf_scr[...] = zeros
    hb_scr[...] = zeros
    cb_scr[...] = zeros

    iota = jax.lax.broadcasted_iota(jnp.int32, (BATCH, INPUT_DIM), 1)

    def step_dir(ids, h, c, m_tab, whh):
        onehot = (ids == iota).astype(f32)
        gates = (jnp.dot(onehot, m_tab, preferred_element_type=f32)
                 + jnp.dot(h, whh, preferred_element_type=f32))
        i = jax.nn.sigmoid(gates[:, 0:HIDDEN_DIM])
        f = jax.nn.sigmoid(gates[:, HIDDEN_DIM:2 * HIDDEN_DIM])
        g = jnp.tanh(gates[:, 2 * HIDDEN_DIM:3 * HIDDEN_DIM])
        o = jax.nn.sigmoid(gates[:, 3 * HIDDEN_DIM:])
        c = f * c + i * g
        h = o * jnp.tanh(c)
        return h, c

    mf = mf_scr[...]
    mb = mb_scr[...]
    whhf = whhf_ref[...]
    whhb = whhb_ref[...]

    def body(t, _):
        ids_f = urls_ref[pl.ds(t * BATCH, BATCH), :]
        ids_b = urls_ref[pl.ds((SEQ - 1 - t) * BATCH, BATCH), :]
        hf, cf = step_dir(ids_f, hf_scr[...], cf_scr[...], mf, whhf)
        hb, cb = step_dir(ids_b, hb_scr[...], cb_scr[...], mb, whhb)
        hf_scr[...] = hf
        cf_scr[...] = cf
        hb_scr[...] = hb
        cb_scr[...] = cb
        return 0

    jax.lax.fori_loop(0, SEQ, body, 0)

    hidden = jnp.concatenate([hf_scr[...], hb_scr[...]], axis=1)
    aux_ref[...] = jnp.dot(hidden, fcw_ref[...], preferred_element_type=f32) + fcb_ref[...]
    o1 = jnp.dot(hidden, fc1w_ref[...], preferred_element_type=f32) + fc1b_ref[...]
    o2 = jnp.dot(o1, fc2w_ref[...], preferred_element_type=f32) + fc2b_ref[...]
    out_ref[...] = jnp.dot(o2, fc3w_ref[...], preferred_element_type=f32) + fc3b_ref[...]


@functools.partial(jax.jit, static_argnames=("interpret",))
def _run(urls, emb_table, W_ih_f, W_hh_f, b_f, W_ih_b, W_hh_b, b_b,
         fc_w, fc_b, fc1_w, fc1_b, fc2_w, fc2_b, fc3_w, fc3_b,
         interpret=False):
    urls_flat = urls.T.reshape(SEQ * BATCH, 1).astype(jnp.int32)
    f32 = jnp.float32
    args = (
        urls_flat,
        emb_table,
        W_ih_f.T, W_hh_f.T, b_f.reshape(1, H4),
        W_ih_b.T, W_hh_b.T, b_b.reshape(1, H4),
        fc_w.T, fc_b.reshape(1, 1),
        fc1_w.T, fc1_b.reshape(1, H4),
        fc2_w.T, fc2_b.reshape(1, 2 * HIDDEN_DIM),
        fc3_w.T, fc3_b.reshape(1, 2),
    )
    out, aux = pl.pallas_call(
        _lstm_kernel,
        out_shape=(
            jax.ShapeDtypeStruct((BATCH, 2), f32),
            jax.ShapeDtypeStruct((BATCH, 1), f32),
        ),
        scratch_shapes=[
            pltpu.VMEM((INPUT_DIM, H4), f32),
            pltpu.VMEM((INPUT_DIM, H4), f32),
            pltpu.VMEM((BATCH, HIDDEN_DIM), f32),
            pltpu.VMEM((BATCH, HIDDEN_DIM), f32),
            pltpu.VMEM((BATCH, HIDDEN_DIM), f32),
            pltpu.VMEM((BATCH, HIDDEN_DIM), f32),
        ],
        interpret=interpret,
    )(*args)
    return out, aux[:, 0]


def kernel(urls, emb_table, W_ih_f, W_hh_f, b_f, W_ih_b, W_hh_b, b_b,
           fc_w, fc_b, fc1_w, fc1_b, fc2_w, fc2_b, fc3_w, fc3_b):
    return _run(urls, emb_table, W_ih_f, W_hh_f, b_f, W_ih_b, W_hh_b, b_b,
                fc_w, fc_b, fc1_w, fc1_b, fc2_w, fc2_b, fc3_w, fc3_b)


# speedup vs baseline: 3.2411x; 1.0013x over previous
import functools

import jax
import jax.numpy as jnp
from jax.experimental import pallas as pl
from jax.experimental.pallas import tpu as pltpu

INPUT_DIM = 256
EMB_DIM = 128
HIDDEN_DIM = 512
BATCH = 128
SEQ = 200
H4 = 4 * HIDDEN_DIM


def _lstm_kernel(urls_ref, emb_ref, wihf_ref, whhf_ref, bf_ref,
                 wihb_ref, whhb_ref, bb_ref,
                 fcw_ref, fcb_ref, fc1w_ref, fc1b_ref,
                 fc2w_ref, fc2b_ref, fc3w_ref, fc3b_ref,
                 out_ref, aux_ref,
                 mf_scr, mb_scr, hf_scr, cf_scr, hb_scr, cb_scr):
    f32 = jnp.float32
    bf16 = jnp.bfloat16

    emb = emb_ref[...]
    mf_scr[...] = (jnp.dot(emb, wihf_ref[...], preferred_element_type=f32)
                   + bf_ref[...]).astype(bf16)
    mb_scr[...] = (jnp.dot(emb, wihb_ref[...], preferred_element_type=f32)
                   + bb_ref[...]).astype(bf16)

    hf_scr[...] = jnp.zeros((BATCH, HIDDEN_DIM), bf16)
    hb_scr[...] = jnp.zeros((BATCH, HIDDEN_DIM), bf16)
    cf_scr[...] = jnp.zeros((BATCH, HIDDEN_DIM), f32)
    cb_scr[...] = jnp.zeros((BATCH, HIDDEN_DIM), f32)

    iota = jax.lax.broadcasted_iota(jnp.int32, (BATCH, INPUT_DIM), 1)

    def step_dir(ids, h, c, m_tab, whh):
        onehot = (ids == iota).astype(bf16)
        gates = (jnp.dot(onehot, m_tab, preferred_element_type=f32)
                 + jnp.dot(h, whh, preferred_element_type=f32))
        i = jax.nn.sigmoid(gates[:, 0:HIDDEN_DIM])
        f = jax.nn.sigmoid(gates[:, HIDDEN_DIM:2 * HIDDEN_DIM])
        g = jnp.tanh(gates[:, 2 * HIDDEN_DIM:3 * HIDDEN_DIM])
        o = jax.nn.sigmoid(gates[:, 3 * HIDDEN_DIM:])
        c = f * c + i * g
        h = (o * jnp.tanh(c)).astype(bf16)
        return h, c

    mf = mf_scr[...]
    mb = mb_scr[...]
    whhf = whhf_ref[...]
    whhb = whhb_ref[...]

    def body(t, _):
        ids_f = urls_ref[pl.ds(t * BATCH, BATCH), :]
        ids_b = urls_ref[pl.ds((SEQ - 1 - t) * BATCH, BATCH), :]
        hf, cf = step_dir(ids_f, hf_scr[...], cf_scr[...], mf, whhf)
        hb, cb = step_dir(ids_b, hb_scr[...], cb_scr[...], mb, whhb)
        hf_scr[...] = hf
        cf_scr[...] = cf
        hb_scr[...] = hb
        cb_scr[...] = cb
        return 0

    jax.lax.fori_loop(0, SEQ, body, 0)

    hidden = jnp.concatenate([hf_scr[...], hb_scr[...]], axis=1).astype(f32)
    aux_ref[...] = jnp.dot(hidden, fcw_ref[...], preferred_element_type=f32) + fcb_ref[...]
    o1 = jnp.dot(hidden, fc1w_ref[...], preferred_element_type=f32) + fc1b_ref[...]
    o2 = jnp.dot(o1, fc2w_ref[...], preferred_element_type=f32) + fc2b_ref[...]
    out_ref[...] = jnp.dot(o2, fc3w_ref[...], preferred_element_type=f32) + fc3b_ref[...]


@functools.partial(jax.jit, static_argnames=("interpret",))
def _run(urls, emb_table, W_ih_f, W_hh_f, b_f, W_ih_b, W_hh_b, b_b,
         fc_w, fc_b, fc1_w, fc1_b, fc2_w, fc2_b, fc3_w, fc3_b,
         interpret=False):
    urls_flat = urls.T.reshape(SEQ * BATCH, 1).astype(jnp.int32)
    f32 = jnp.float32
    args = (
        urls_flat,
        emb_table,
        W_ih_f.T, W_hh_f.T.astype(jnp.bfloat16), b_f.reshape(1, H4),
        W_ih_b.T, W_hh_b.T.astype(jnp.bfloat16), b_b.reshape(1, H4),
        fc_w.T, fc_b.reshape(1, 1),
        fc1_w.T, fc1_b.reshape(1, H4),
        fc2_w.T, fc2_b.reshape(1, 2 * HIDDEN_DIM),
        fc3_w.T, fc3_b.reshape(1, 2),
    )
    out, aux = pl.pallas_call(
        _lstm_kernel,
        out_shape=(
            jax.ShapeDtypeStruct((BATCH, 2), f32),
            jax.ShapeDtypeStruct((BATCH, 1), f32),
        ),
        scratch_shapes=[
            pltpu.VMEM((INPUT_DIM, H4), jnp.bfloat16),
            pltpu.VMEM((INPUT_DIM, H4), jnp.bfloat16),
            pltpu.VMEM((BATCH, HIDDEN_DIM), jnp.bfloat16),
            pltpu.VMEM((BATCH, HIDDEN_DIM), f32),
            pltpu.VMEM((BATCH, HIDDEN_DIM), jnp.bfloat16),
            pltpu.VMEM((BATCH, HIDDEN_DIM), f32),
        ],
        interpret=interpret,
    )(*args)
    return out, aux[:, 0]


def kernel(urls, emb_table, W_ih_f, W_hh_f, b_f, W_ih_b, W_hh_b, b_b,
           fc_w, fc_b, fc1_w, fc1_b, fc2_w, fc2_b, fc3_w, fc3_b):
    return _run(urls, emb_table, W_ih_f, W_hh_f, b_f, W_ih_b, W_hh_b, b_b,
                fc_w, fc_b, fc1_w, fc1_b, fc2_w, fc2_b, fc3_w, fc3_b)


# speedup vs baseline: 3.4058x; 1.0508x over previous
import functools

import jax
import jax.numpy as jnp
from jax.experimental import pallas as pl
from jax.experimental.pallas import tpu as pltpu

INPUT_DIM = 256
EMB_DIM = 128
HIDDEN_DIM = 512
BATCH = 128
SEQ = 200
H4 = 4 * HIDDEN_DIM
CHUNK = 10


def _lstm_kernel(urls_ref, emb_ref, wihf_ref, whhf_ref, bf_ref,
                 wihb_ref, whhb_ref, bb_ref,
                 fcw_ref, fcb_ref, fc1w_ref, fc1b_ref,
                 fc2w_ref, fc2b_ref, fc3w_ref, fc3b_ref,
                 out_ref, aux_ref,
                 mf_scr, mb_scr, hf_scr, cf_scr, hb_scr, cb_scr,
                 gif_scr, gib_scr):
    f32 = jnp.float32
    bf16 = jnp.bfloat16

    emb = emb_ref[...]
    mf_scr[...] = (jnp.dot(emb, wihf_ref[...], preferred_element_type=f32)
                   + bf_ref[...]).astype(bf16)
    mb_scr[...] = (jnp.dot(emb, wihb_ref[...], preferred_element_type=f32)
                   + bb_ref[...]).astype(bf16)

    hf_scr[...] = jnp.zeros((BATCH, HIDDEN_DIM), bf16)
    hb_scr[...] = jnp.zeros((BATCH, HIDDEN_DIM), bf16)
    cf_scr[...] = jnp.zeros((BATCH, HIDDEN_DIM), f32)
    cb_scr[...] = jnp.zeros((BATCH, HIDDEN_DIM), f32)

    iota = jax.lax.broadcasted_iota(jnp.int32, (CHUNK * BATCH, INPUT_DIM), 1)

    def step_dir(gin, h, c, whh):
        gates = jnp.dot(h, whh, preferred_element_type=f32) + gin.astype(f32)
        i = jax.nn.sigmoid(gates[:, 0:HIDDEN_DIM])
        f = jax.nn.sigmoid(gates[:, HIDDEN_DIM:2 * HIDDEN_DIM])
        g = jnp.tanh(gates[:, 2 * HIDDEN_DIM:3 * HIDDEN_DIM])
        o = jax.nn.sigmoid(gates[:, 3 * HIDDEN_DIM:])
        c = f * c + i * g
        h = (o * jnp.tanh(c)).astype(bf16)
        return h, c

    mf = mf_scr[...]
    mb = mb_scr[...]
    whhf = whhf_ref[...]
    whhb = whhb_ref[...]

    def chunk_body(k, _):
        ids_f = urls_ref[pl.ds(k * CHUNK * BATCH, CHUNK * BATCH), :]
        ids_b = urls_ref[pl.ds((SEQ - (k + 1) * CHUNK) * BATCH, CHUNK * BATCH), :]
        gif_scr[...] = jnp.dot((ids_f == iota).astype(bf16), mf,
                               preferred_element_type=f32).astype(bf16)
        gib_scr[...] = jnp.dot((ids_b == iota).astype(bf16), mb,
                               preferred_element_type=f32).astype(bf16)

        def body(j, _):
            gf = gif_scr[pl.ds(j * BATCH, BATCH), :]
            gb = gib_scr[pl.ds((CHUNK - 1 - j) * BATCH, BATCH), :]
            hf, cf = step_dir(gf, hf_scr[...], cf_scr[...], whhf)
            hb, cb = step_dir(gb, hb_scr[...], cb_scr[...], whhb)
            hf_scr[...] = hf
            cf_scr[...] = cf
            hb_scr[...] = hb
            cb_scr[...] = cb
            return 0

        jax.lax.fori_loop(0, CHUNK, body, 0)
        return 0

    jax.lax.fori_loop(0, SEQ // CHUNK, chunk_body, 0)

    hidden = jnp.concatenate([hf_scr[...], hb_scr[...]], axis=1).astype(f32)
    aux_ref[...] = jnp.dot(hidden, fcw_ref[...], preferred_element_type=f32) + fcb_ref[...]
    o1 = jnp.dot(hidden, fc1w_ref[...], preferred_element_type=f32) + fc1b_ref[...]
    o2 = jnp.dot(o1, fc2w_ref[...], preferred_element_type=f32) + fc2b_ref[...]
    out_ref[...] = jnp.dot(o2, fc3w_ref[...], preferred_element_type=f32) + fc3b_ref[...]


@functools.partial(jax.jit, static_argnames=("interpret",))
def _run(urls, emb_table, W_ih_f, W_hh_f, b_f, W_ih_b, W_hh_b, b_b,
         fc_w, fc_b, fc1_w, fc1_b, fc2_w, fc2_b, fc3_w, fc3_b,
         interpret=False):
    urls_flat = urls.T.reshape(SEQ * BATCH, 1).astype(jnp.int32)
    f32 = jnp.float32
    args = (
        urls_flat,
        emb_table,
        W_ih_f.T, W_hh_f.T.astype(jnp.bfloat16), b_f.reshape(1, H4),
        W_ih_b.T, W_hh_b.T.astype(jnp.bfloat16), b_b.reshape(1, H4),
        fc_w.T, fc_b.reshape(1, 1),
        fc1_w.T, fc1_b.reshape(1, H4),
        fc2_w.T, fc2_b.reshape(1, 2 * HIDDEN_DIM),
        fc3_w.T, fc3_b.reshape(1, 2),
    )
    out, aux = pl.pallas_call(
        _lstm_kernel,
        out_shape=(
            jax.ShapeDtypeStruct((BATCH, 2), f32),
            jax.ShapeDtypeStruct((BATCH, 1), f32),
        ),
        scratch_shapes=[
            pltpu.VMEM((INPUT_DIM, H4), jnp.bfloat16),
            pltpu.VMEM((INPUT_DIM, H4), jnp.bfloat16),
            pltpu.VMEM((BATCH, HIDDEN_DIM), jnp.bfloat16),
            pltpu.VMEM((BATCH, HIDDEN_DIM), f32),
            pltpu.VMEM((BATCH, HIDDEN_DIM), jnp.bfloat16),
            pltpu.VMEM((BATCH, HIDDEN_DIM), f32),
            pltpu.VMEM((CHUNK * BATCH, H4), jnp.bfloat16),
            pltpu.VMEM((CHUNK * BATCH, H4), jnp.bfloat16),
        ],
        interpret=interpret,
    )(*args)
    return out, aux[:, 0]


def kernel(urls, emb_table, W_ih_f, W_hh_f, b_f, W_ih_b, W_hh_b, b_b,
           fc_w, fc_b, fc1_w, fc1_b, fc2_w, fc2_b, fc3_w, fc3_b):
    return _run(urls, emb_table, W_ih_f, W_hh_f, b_f, W_ih_b, W_hh_b, b_b,
                fc_w, fc_b, fc1_w, fc1_b, fc2_w, fc2_b, fc3_w, fc3_b)
